# Initial kernel scaffold; baseline (speedup 1.0000x reference)
import jax
import jax.numpy as jnp
from jax.experimental import pallas as pl


def kernel(x, edge_attr, edge_index, batch, Wl, bl, Wr, We, be, W1, b1, W2, b2):
    raise NotImplementedError("write your pallas kernel here")


# speedup vs baseline: 3.4700x; 3.4700x over previous
import functools

import jax
import jax.numpy as jnp
from jax import lax
from jax.experimental import pallas as pl
from jax.experimental.pallas import tpu as pltpu
from jax.experimental.pallas import tpu_sc as plsc

_N = 10000
_NP = 10240
_E = 320000
_EP = 327680
_NDEG = 11
_H = 32
_NG = 64
_NEG = -3.0e38

_mesh = functools.partial(
    plsc.VectorSubcoreMesh, core_axis_name="c", subcore_axis_name="s"
)

_SC_PARAMS = pltpu.CompilerParams(needs_layout_passes=False,
                                  use_tc_tiling_on_sc=False)


def _iota16():
    return lax.iota(jnp.int32, 16)


def _sc_pass_a(srcm, dstm, b2, negc, zer1):

    def body(srcm_h, dstm_h, b2_h, negc_h, zer1_h, dego_h, maxo_h,
             srcb, dstb, brows, maxacc, deghist, sem):
        cid = lax.axis_index("c")
        sid = lax.axis_index("s")
        eg = cid * 4 + sid // 4
        cg = sid % 4
        hc = cg // 2
        side = cg % 2

        pltpu.sync_copy(negc_h, maxacc)
        pltpu.sync_copy(zer1_h, deghist)

        base_row = eg * 320
        lanes = _iota16()
        valid = (lanes >= side * 8) & (lanes < side * 8 + 8)
        av = jnp.where(valid, (lanes - side * 8) * _NP, 0)
        lane0 = lanes == side * 8
        ones = jnp.full((16,), 1.0, jnp.float32)
        row_off = hc * _NP

        @pl.loop(0, 80)
        def _chunk(k):
            r0 = base_row + k * 4
            pltpu.sync_copy(srcm_h.at[pl.ds(r0, 4)], srcb)
            pltpu.sync_copy(dstm_h.at[pl.ds(r0, 4)], dstb)

            @pl.loop(0, 32)
            def _mkidx(i):
                j = i // 8
                o = (i % 8) * 16
                srcb[j, pl.ds(o, 16)] = srcb[j, pl.ds(o, 16)] + row_off

            cps = [pltpu.async_copy(
                b2_h.at[srcb.at[j]], brows.at[pl.ds(j * 128, 128)], sem)
                for j in range(4)]
            for cp in cps:
                cp.wait()

            @pl.loop(0, 32)
            def _grp(i):
                j = i // 8
                o = (i % 8) * 16
                dst16 = dstb[j, pl.ds(o, 16)]
                for i2 in range(16):
                    d = dst16[i2]
                    val = brows[i * 16 + i2, :]
                    addr = av + d
                    cur = plsc.load_gather(maxacc, [addr])
                    plsc.store_scatter(maxacc, [addr],
                                       jnp.maximum(cur, val), mask=valid)
                    plsc.addupdate_scatter(
                        deghist, [jnp.broadcast_to(d, (16,))], ones,
                        mask=lane0)

        gwid = cid * 16 + sid
        pltpu.sync_copy(maxacc, maxo_h.at[gwid])
        pltpu.sync_copy(deghist, dego_h.at[gwid])

    f = pl.kernel(
        body,
        out_type=(
            jax.ShapeDtypeStruct((32, _NP), jnp.float32),
            jax.ShapeDtypeStruct((32, 8 * _NP), jnp.float32),
        ),
        mesh=_mesh(),
        scratch_types=[
            pltpu.VMEM((4, 128), jnp.int32),
            pltpu.VMEM((4, 128), jnp.int32),
            pltpu.VMEM((512, 16), jnp.float32),
            pltpu.VMEM((8 * _NP,), jnp.float32),
            pltpu.VMEM((_NP,), jnp.float32),
            pltpu.SemaphoreType.DMA,
        ],
        name="sc_deg_segmax",
        compiler_params=_SC_PARAMS,
    )
    return f(srcm, dstm, b2, negc, zer1)


def _sc_gather_segsum(srcm, dstm, table, z32, degc=None):
    with_deg = degc is not None

    def body(*refs):
        if with_deg:
            (srcm_h, dstm_h, tab_h, z32_h, degc_h, out_h,
             srcb, dstb, degv, rows, acc, sem) = refs
        else:
            (srcm_h, dstm_h, tab_h, z32_h, out_h,
             srcb, dstb, rows, acc, sem) = refs
        cid = lax.axis_index("c")
        sid = lax.axis_index("s")
        wid = cid * 16 + sid

        if with_deg:
            pltpu.sync_copy(degc_h, degv)

        @pl.when(sid == 0)
        def _init():
            pltpu.sync_copy(z32_h, acc)

        plsc.subcore_barrier()

        base_row = wid * 80

        @pl.loop(0, 20)
        def _chunk(k):
            r0 = base_row + k * 4
            pltpu.sync_copy(srcm_h.at[pl.ds(r0, 4)], srcb)
            pltpu.sync_copy(dstm_h.at[pl.ds(r0, 4)], dstb)

            if with_deg:
                @pl.loop(0, 32)
                def _mkidx(i):
                    j = i // 8
                    o = (i % 8) * 16
                    dst16 = dstb[j, pl.ds(o, 16)]
                    d16 = plsc.load_gather(degv, [dst16])
                    srcb[j, pl.ds(o, 16)] = (
                        srcb[j, pl.ds(o, 16)] + d16 * _NP)

            cps = [pltpu.async_copy(
                tab_h.at[srcb.at[j]], rows.at[pl.ds(j * 128, 128)], sem)
                for j in range(4)]
            for cp in cps:
                cp.wait()
            for j in range(4):
                pltpu.sync_copy(
                    rows.at[pl.ds(j * 128, 128)], acc.at[dstb.at[j]],
                    add=True)

        plsc.subcore_barrier()
        rs = sid * 640
        pltpu.sync_copy(acc.at[pl.ds(rs, 640)],
                        out_h.at[cid, pl.ds(rs, 640)])

    scratch = [
        pltpu.VMEM((4, 128), jnp.int32),
        pltpu.VMEM((4, 128), jnp.int32),
    ]
    if with_deg:
        scratch.append(pltpu.VMEM((_NP,), jnp.int32))
    scratch += [
        pltpu.VMEM((512, 32), jnp.float32),
        pltpu.VMEM_SHARED((_NP, 32), jnp.float32),
        pltpu.SemaphoreType.DMA,
    ]
    f = pl.kernel(
        body,
        out_type=jax.ShapeDtypeStruct((2, _NP, 32), jnp.float32),
        mesh=_mesh(),
        scratch_types=scratch,
        name="sc_gather_segsum" + ("_deg" if with_deg else ""),
        compiler_params=_SC_PARAMS,
    )
    args = (srcm, dstm, table, z32) + ((degc,) if with_deg else ())
    return f(*args)


def _tc_ab(ean, wea, web):
    def body(ean_ref, wea_ref, web_ref, a_ref, b2_ref):
        a_ref[...] = jnp.dot(ean_ref[...], wea_ref[...],
                             preferred_element_type=jnp.float32)
        b = jnp.dot(ean_ref[...], web_ref[...],
                    preferred_element_type=jnp.float32)
        b2_ref[0] = b[:, :16]
        b2_ref[1] = b[:, 16:]

    return pl.pallas_call(
        body,
        grid=(8,),
        in_specs=[
            pl.BlockSpec((1280, 16), lambda i: (i, 0)),
            pl.BlockSpec((16, 32), lambda i: (0, 0)),
            pl.BlockSpec((16, 32), lambda i: (0, 0)),
        ],
        out_specs=[
            pl.BlockSpec((1280, 32), lambda i: (i, 0)),
            pl.BlockSpec((2, 1280, 16), lambda i: (0, i, 0)),
        ],
        out_shape=[
            jax.ShapeDtypeStruct((_NP, 32), jnp.float32),
            jax.ShapeDtypeStruct((2, _NP, 16), jnp.float32),
        ],
    )(ean, wea, web)


def _tc_p(xp, wl):
    def body(x_ref, wl_ref, p_ref):
        p_ref[0] = jnp.dot(x_ref[...], wl_ref[0],
                           preferred_element_type=jnp.float32)

    return pl.pallas_call(
        body,
        grid=(10, _NDEG),
        in_specs=[
            pl.BlockSpec((1024, 128), lambda i, d: (i, 0)),
            pl.BlockSpec((1, 128, 32), lambda i, d: (d, 0, 0)),
        ],
        out_specs=pl.BlockSpec((1, 1024, 32), lambda i, d: (d, i, 0)),
        out_shape=jax.ShapeDtypeStruct((_NDEG, _NP, 32), jnp.float32),
    )(xp, wl)


def _tc_deg_ea(dego, maxo, a, be2):
    def body(dego_ref, maxo_ref, a_ref, be_ref, degc_ref, ea_ref, w2_ref):
        deg2 = jnp.sum(dego_ref[...], axis=0, keepdims=True) * 0.25
        degcol = deg2.T
        mv = maxo_ref[...].reshape(2, 4, 4, 8, 1280)
        m_t = jnp.max(mv, axis=(0, 1)).reshape(32, 1280)
        m = m_t.T
        ea = jnp.where(degcol > 0.0, a_ref[...] + be_ref[...] + m, 0.0)
        degc_ref[...] = jnp.minimum(deg2, 10.0).astype(jnp.int32)
        ea_ref[...] = ea
        w2_ref[...] = degcol * jax.nn.relu(ea)

    return pl.pallas_call(
        body,
        grid=(8,),
        in_specs=[
            pl.BlockSpec((32, 1280), lambda i: (0, i)),
            pl.BlockSpec((32, 8, 1280), lambda i: (0, 0, i)),
            pl.BlockSpec((1280, 32), lambda i: (i, 0)),
            pl.BlockSpec((1, 32), lambda i: (0, 0)),
        ],
        out_specs=[
            pl.BlockSpec((1, 1280), lambda i: (0, i)),
            pl.BlockSpec((1280, 32), lambda i: (i, 0)),
            pl.BlockSpec((1280, 32), lambda i: (i, 0)),
        ],
        out_shape=[
            jax.ShapeDtypeStruct((1, _NP), jnp.int32),
            jax.ShapeDtypeStruct((_NP, 32), jnp.float32),
            jax.ShapeDtypeStruct((_NP, 32), jnp.float32),
        ],
    )(dego, maxo, a, be2)


def _tc_mfout(hselo, degc, xp, wrf, blf):
    def body(h_ref, degc_ref, x_ref, wr_ref, bl_ref, emb_ref, xh_ref):
        hs = h_ref[0] + h_ref[1]
        q = jnp.dot(x_ref[...], wr_ref[...],
                    preferred_element_type=jnp.float32)
        degcol = degc_ref[...].T
        sel = jnp.zeros((1024, 32), jnp.float32)
        for d in range(_NDEG):
            sel = jnp.where(degcol == d,
                            q[:, d * 32:(d + 1) * 32] + bl_ref[d][None, :],
                            sel)
        out = hs + sel
        emb_ref[...] = out
        xh_ref[...] = jax.nn.relu(out)

    return pl.pallas_call(
        body,
        grid=(10,),
        in_specs=[
            pl.BlockSpec((2, 1024, 32), lambda i: (0, i, 0)),
            pl.BlockSpec((1, 1024), lambda i: (0, i)),
            pl.BlockSpec((1024, 128), lambda i: (i, 0)),
            pl.BlockSpec((128, 352), lambda i: (0, 0)),
            pl.BlockSpec((_NDEG, 32), lambda i: (0, 0)),
        ],
        out_specs=[
            pl.BlockSpec((1024, 32), lambda i: (i, 0)),
            pl.BlockSpec((1024, 32), lambda i: (i, 0)),
        ],
        out_shape=[
            jax.ShapeDtypeStruct((_NP, 32), jnp.float32),
            jax.ShapeDtypeStruct((_NP, 32), jnp.float32),
        ],
    )(hselo, degc, xp, wrf, blf)


def _tc_pool(s1o, w2, batchp, w1, b1, w2w, b2):
    def body(s1_ref, w2_ref, b_ref, w1_ref, b1_ref, w2w_ref, b2_ref,
             pred_ref, acca, accb):
        i = pl.program_id(0)

        @pl.when(i == 0)
        def _():
            acca[...] = jnp.zeros((_NG, 32), jnp.float32)
            accb[...] = jnp.zeros((_NG, 32), jnp.float32)

        gids = lax.broadcasted_iota(jnp.int32, (_NG, 1280), 0)
        oh = (gids == b_ref[...]).astype(jnp.float32)
        s1 = s1_ref[0] + s1_ref[1]
        acca[...] = acca[...] + jnp.dot(
            oh, s1, preferred_element_type=jnp.float32)
        accb[...] = accb[...] + jnp.dot(
            oh, w2_ref[...], preferred_element_type=jnp.float32)

        @pl.when(i == 7)
        def _():
            h = (jnp.dot(acca[...], w1_ref[:32, :],
                         preferred_element_type=jnp.float32)
                 + jnp.dot(accb[...], w1_ref[32:, :],
                           preferred_element_type=jnp.float32)
                 + b1_ref[...])
            pred_ref[...] = jnp.dot(h, w2w_ref[...],
                                    preferred_element_type=jnp.float32) \
                + b2_ref[...]

    return pl.pallas_call(
        body,
        grid=(8,),
        in_specs=[
            pl.BlockSpec((2, 1280, 32), lambda i: (0, i, 0)),
            pl.BlockSpec((1280, 32), lambda i: (i, 0)),
            pl.BlockSpec((1, 1280), lambda i: (0, i)),
            pl.BlockSpec((64, 32), lambda i: (0, 0)),
            pl.BlockSpec((1, 32), lambda i: (0, 0)),
            pl.BlockSpec((32, 1), lambda i: (0, 0)),
            pl.BlockSpec((1, 1), lambda i: (0, 0)),
        ],
        out_specs=pl.BlockSpec((_NG, 1), lambda i: (0, 0)),
        out_shape=jax.ShapeDtypeStruct((_NG, 1), jnp.float32),
        scratch_shapes=[
            pltpu.VMEM((_NG, 32), jnp.float32),
            pltpu.VMEM((_NG, 32), jnp.float32),
        ],
    )(s1o, w2, batchp, w1, b1, w2w, b2)


def kernel(x, edge_attr, edge_index, batch, Wl, bl, Wr, We, be, W1, b1, W2,
           b2):
    npad = _EP - _E
    pad_src = (jnp.arange(npad, dtype=jnp.int32) * 131 + 7) % _N
    pad_dst = _N + jnp.arange(npad, dtype=jnp.int32) % (_NP - _N)
    src = jnp.concatenate([edge_index[0], pad_src]).reshape(2560, 128)
    dst = jnp.concatenate([edge_index[1], pad_dst]).reshape(2560, 128)

    xp = jnp.zeros((_NP, 128), jnp.float32).at[:_N].set(x)
    ean = jnp.zeros((_NP, 16), jnp.float32).at[:_N].set(edge_attr[:_N])
    batchp = jnp.full((1, _NP), _NG, jnp.int32).at[0, :_N].set(batch)

    wea = We[:16] - We[16:]
    web = We[16:]
    wrf = jnp.transpose(Wr, (1, 0, 2)).reshape(128, 352)
    be2 = be.reshape(1, 32)
    b12 = b1.reshape(1, 32)
    b22 = b2.reshape(1, 1)

    negc = jnp.full((8 * _NP,), _NEG, jnp.float32)
    zer1 = jnp.zeros((_NP,), jnp.float32)
    z32 = jnp.zeros((_NP, 32), jnp.float32)

    a_t, b2 = _tc_ab(ean, wea, web)
    ptab = _tc_p(xp, Wl)

    dego, maxo = _sc_pass_a(src, dst, b2.reshape(2 * _NP, 16), negc, zer1)
    degc, ea, w2 = _tc_deg_ea(dego, maxo.reshape(32, 8, _NP), a_t, be2)

    hselo = _sc_gather_segsum(src, dst, ptab.reshape(_NDEG * _NP, 32), z32,
                              degc=degc.reshape(_NP))
    embn, xh = _tc_mfout(hselo, degc, xp, wrf, bl)

    s1o = _sc_gather_segsum(src, dst, xh, z32)
    pred = _tc_pool(s1o, w2, batchp, W1, b12, W2, b22)

    emb_node = embn[:_N]
    emb_edge = jnp.concatenate(
        [ea[:_N], jnp.zeros((_E - _N, _H), jnp.float32)], axis=0)
    return (pred, emb_node, emb_edge)
